# baseline (speedup 1.0000x reference)
import functools

import jax
import jax.numpy as jnp
from jax import lax
from jax.experimental import pallas as pl
from jax.experimental.pallas import tpu as pltpu
from jax.experimental.pallas import tpu_sc as plsc

_BATCH = 16384
_EMBD = 32
_NW = 32
_BPW = _BATCH // _NW
_CHUNK = 128
_NCH = _BPW // _CHUNK


def _pmf_body(uid_hbm, iid_hbm, ut_hbm, it_hbm, bu_hbm, bi_hbm, out_hbm,
              uid_v, iid_v, u_rows, i_rows, bu_v, bi_v, out_v, sem):
    wid = lax.axis_index("s") * 2 + lax.axis_index("c")

    pltpu.sync_copy(uid_hbm.at[wid], uid_v)
    pltpu.sync_copy(iid_hbm.at[wid], iid_v)

    copies = []
    for j in range(_NCH):
        sl = pl.ds(j * _CHUNK, _CHUNK)
        copies.append(pltpu.async_copy(ut_hbm.at[uid_v.at[j]], u_rows.at[sl], sem))
        copies.append(pltpu.async_copy(it_hbm.at[iid_v.at[j]], i_rows.at[sl], sem))
        copies.append(pltpu.async_copy(bu_hbm.at[uid_v.at[j]], bu_v.at[sl], sem))
        copies.append(pltpu.async_copy(bi_hbm.at[iid_v.at[j]], bi_v.at[sl], sem))
    for c in copies:
        c.wait()

    riota = lax.iota(jnp.int32, 16)

    def chunk_body(c, carry):
        base = c * 16
        acc = bu_v[pl.ds(base, 16)] + bi_v[pl.ds(base, 16)]
        rows = riota + base
        for d in range(_EMBD):
            cols = jnp.full((16,), d, jnp.int32)
            u = plsc.load_gather(u_rows, [rows, cols])
            v = plsc.load_gather(i_rows, [rows, cols])
            acc = acc + u * v
        out_v[pl.ds(base, 16)] = acc
        return carry

    lax.fori_loop(0, _BPW // 16, chunk_body, 0)

    pltpu.sync_copy(out_v, out_hbm.at[wid])


@jax.jit
def _pmf(uid, iid, user_table, item_table, b_users, b_items):
    mesh = plsc.VectorSubcoreMesh(core_axis_name="c", subcore_axis_name="s")
    kfn = pl.kernel(
        _pmf_body,
        out_type=jax.ShapeDtypeStruct((_NW, _BPW), jnp.float32),
        mesh=mesh,
        scratch_types=[
            pltpu.VMEM((_NCH, _CHUNK), jnp.int32),
            pltpu.VMEM((_NCH, _CHUNK), jnp.int32),
            pltpu.VMEM((_BPW, _EMBD), jnp.float32),
            pltpu.VMEM((_BPW, _EMBD), jnp.float32),
            pltpu.VMEM((_BPW,), jnp.float32),
            pltpu.VMEM((_BPW,), jnp.float32),
            pltpu.VMEM((_BPW,), jnp.float32),
            pltpu.SemaphoreType.DMA,
        ],
        compiler_params=pltpu.CompilerParams(
            needs_layout_passes=False, use_tc_tiling_on_sc=False),
        name="pmf_sc",
    )
    return kfn(uid, iid, user_table, item_table, b_users, b_items)


def kernel(user_review, item_review, uid, iid, user_table, item_table,
           b_users, b_items, b_0):
    del user_review, item_review
    uid = uid.astype(jnp.int32).reshape(_NW, _NCH, _CHUNK)
    iid = iid.astype(jnp.int32).reshape(_NW, _NCH, _CHUNK)
    out = _pmf(uid, iid, user_table, item_table,
               b_users.reshape(-1), b_items.reshape(-1))
    return out.reshape(_BATCH) + b_0[0]
